# Initial kernel scaffold; baseline (speedup 1.0000x reference)
import jax
import jax.numpy as jnp
from jax.experimental import pallas as pl


def kernel(x, edge_index, batch, W1, b1, W2, b2, Wl, bl):
    raise NotImplementedError("write your pallas kernel here")


# speedup vs baseline: 6.7529x; 6.7529x over previous
import functools

import jax
import jax.numpy as jnp
from jax import lax
from jax.experimental import pallas as pl
from jax.experimental.pallas import tpu as pltpu
from jax.experimental.pallas import tpu_sc as plsc

_NC = 2
_NS = 16
_NW = _NC * _NS
_BLK = 128
_RB = 400


def _cdiv(a, b):
    return (a + b - 1) // b


def _make_sc_deg(R, RPT, NBT):
    mesh = plsc.VectorSubcoreMesh(core_axis_name="c", subcore_axis_name="s")

    @functools.partial(
        pl.kernel,
        out_type=jax.ShapeDtypeStruct((_NC, R, 16), jnp.float32),
        mesh=mesh,
        scratch_types=[
            pltpu.VMEM((NBT, _BLK), jnp.int32),
            pltpu.VMEM((_BLK, 16), jnp.float32),
            pltpu.VMEM((RPT, 16), jnp.float32),
            pltpu.VMEM_SHARED((R, 16), jnp.float32),
        ],
        compiler_params=pltpu.CompilerParams(use_tc_tiling_on_sc=False),
    )
    def deg_kernel(dst_hbm, out_hbm, didx, ones_b, bounce, acc):
        c = lax.axis_index("c")
        s = lax.axis_index("s")
        wid = c * _NS + s
        pltpu.sync_copy(dst_hbm.at[pl.ds(wid * NBT, NBT)], didx)

        ov = jnp.ones((16,), jnp.float32)
        zv = jnp.zeros((16,), jnp.float32)

        def fill(r, _):
            ones_b[r, :] = ov
            return 0

        lax.fori_loop(0, _BLK, fill, 0)

        def zero(r, _):
            bounce[r, :] = zv
            return 0

        lax.fori_loop(0, RPT, zero, 0)
        pltpu.sync_copy(bounce, acc.at[pl.ds(s * RPT, RPT)])
        plsc.subcore_barrier()

        def body(i, _):
            pltpu.sync_copy(ones_b, acc.at[didx.at[i]], add=True)
            return 0

        lax.fori_loop(0, NBT, body, 0)
        plsc.subcore_barrier()

        pltpu.sync_copy(acc.at[pl.ds(s * RPT, RPT)], bounce)
        pltpu.sync_copy(bounce, out_hbm.at[c].at[pl.ds(s * RPT, RPT)])

    return deg_kernel


def _make_sc_agg(R, RPT, NBT, D):
    mesh = plsc.VectorSubcoreMesh(core_axis_name="c", subcore_axis_name="s")

    @functools.partial(
        pl.kernel,
        out_type=jax.ShapeDtypeStruct((_NC, R, D), jnp.float32),
        mesh=mesh,
        scratch_types=[
            pltpu.VMEM((NBT, _BLK), jnp.int32),
            pltpu.VMEM((NBT, _BLK), jnp.int32),
            pltpu.VMEM((_BLK, D), jnp.float32),
            pltpu.VMEM((RPT, D), jnp.float32),
            pltpu.VMEM_SHARED((R, D), jnp.float32),
            pltpu.SemaphoreType.DMA,
        ],
        compiler_params=pltpu.CompilerParams(use_tc_tiling_on_sc=False),
    )
    def agg_kernel(g_hbm, src_hbm, dst_hbm, out_hbm, sidx, didx, rows, bounce,
                   acc, sem):
        c = lax.axis_index("c")
        s = lax.axis_index("s")
        wid = c * _NS + s
        pltpu.sync_copy(src_hbm.at[pl.ds(wid * NBT, NBT)], sidx)
        pltpu.sync_copy(dst_hbm.at[pl.ds(wid * NBT, NBT)], didx)

        zv = jnp.zeros((16,), jnp.float32)
        nsub = D // 16

        def zero(r, _):
            for j in range(nsub):
                bounce[r, pl.ds(j * 16, 16)] = zv
            return 0

        lax.fori_loop(0, RPT, zero, 0)
        pltpu.sync_copy(bounce, acc.at[pl.ds(s * RPT, RPT)])
        plsc.subcore_barrier()

        def body(i, _):
            pltpu.async_copy(g_hbm.at[sidx.at[i]], rows, sem).wait()
            pltpu.sync_copy(rows, acc.at[didx.at[i]], add=True)
            return 0

        lax.fori_loop(0, NBT, body, 0)
        plsc.subcore_barrier()

        pltpu.sync_copy(acc.at[pl.ds(s * RPT, RPT)], bounce)
        pltpu.sync_copy(bounce, out_hbm.at[c].at[pl.ds(s * RPT, RPT)])

    return agg_kernel


def _mm_body(x_ref, w_ref, o_ref):
    o_ref[...] = jnp.dot(x_ref[...], w_ref[...],
                         preferred_element_type=jnp.float32)


def _scale_body(dp_ref, h0_ref, g_ref, dis_ref):
    deg = dp_ref[0, :, 0:1] + dp_ref[1, :, 0:1]
    dis = lax.rsqrt(deg)
    dis_ref[...] = dis
    g_ref[...] = h0_ref[...] * dis


def _layer2_body(p_ref, dis_ref, b1_ref, w2_ref, g2_ref):
    dis = dis_ref[...]
    h1 = jnp.maximum((p_ref[0] + p_ref[1]) * dis + b1_ref[...], 0.0)
    g2_ref[...] = jnp.dot(h1, w2_ref[...],
                          preferred_element_type=jnp.float32) * dis


def _final_body(p_ref, dis_ref, b2_ref, batch_ref, wl_ref, bl_ref, o_ref,
                acc, cnt):
    i = pl.program_id(0)

    @pl.when(i == 0)
    def _():
        acc[...] = jnp.zeros_like(acc)
        cnt[...] = jnp.zeros_like(cnt)

    h2 = jnp.maximum((p_ref[0] + p_ref[1]) * dis_ref[...] + b2_ref[...], 0.0)
    gids = batch_ref[...]
    oh = (gids == lax.broadcasted_iota(jnp.int32, (1, 64), 1))
    oh = oh.astype(jnp.float32)
    acc[...] += lax.dot_general(oh, h2, (((0,), (0,)), ((), ())),
                                preferred_element_type=jnp.float32)
    cnt[...] += lax.dot_general(oh, jnp.ones((oh.shape[0], 1), jnp.float32),
                                (((0,), (0,)), ((), ())),
                                preferred_element_type=jnp.float32)

    @pl.when(i == pl.num_programs(0) - 1)
    def _():
        pooled = jnp.dot(acc[...], wl_ref[...],
                         preferred_element_type=jnp.float32)
        o_ref[...] = pooled / jnp.maximum(cnt[...], 1.0) + bl_ref[...]


def kernel(x, edge_index, batch, W1, b1, W2, b2, Wl, bl):
    n, d_in = x.shape
    d_hid = W1.shape[1]
    e = edge_index.shape[1]
    g_graphs = 64

    e_tot = e + n
    nbt = _cdiv(e_tot, _NW * _BLK * 8) * 8
    e_pad = nbt * _NW * _BLK
    rpt = _cdiv(n + 1, _NS * 8) * 8
    R = rpt * _NS

    loop = jnp.arange(n, dtype=jnp.int32)
    pad = e_pad - e_tot
    src_pad = jnp.concatenate(
        [edge_index[0], loop, jnp.zeros((pad,), jnp.int32)]).reshape(-1, _BLK)
    dst_pad = jnp.concatenate(
        [edge_index[1], loop,
         jnp.full((pad,), n, jnp.int32)]).reshape(-1, _BLK)

    deg_parts = _make_sc_deg(R, rpt, nbt)(dst_pad)

    grid = n // _RB
    h0 = pl.pallas_call(
        _mm_body,
        grid=(grid,),
        in_specs=[
            pl.BlockSpec((_RB, d_in), lambda i: (i, 0)),
            pl.BlockSpec((d_in, d_hid), lambda i: (0, 0)),
        ],
        out_specs=pl.BlockSpec((_RB, d_hid), lambda i: (i, 0)),
        out_shape=jax.ShapeDtypeStruct((n, d_hid), jnp.float32),
    )(x, W1)

    g1, dis = pl.pallas_call(
        _scale_body,
        grid=(grid,),
        in_specs=[
            pl.BlockSpec((2, _RB, 16), lambda i: (0, i, 0)),
            pl.BlockSpec((_RB, d_hid), lambda i: (i, 0)),
        ],
        out_specs=[
            pl.BlockSpec((_RB, d_hid), lambda i: (i, 0)),
            pl.BlockSpec((_RB, 1), lambda i: (i, 0)),
        ],
        out_shape=[
            jax.ShapeDtypeStruct((n, d_hid), jnp.float32),
            jax.ShapeDtypeStruct((n, 1), jnp.float32),
        ],
    )(deg_parts, h0)

    agg = _make_sc_agg(R, rpt, nbt, d_hid)
    p1 = agg(g1, src_pad, dst_pad)

    g2 = pl.pallas_call(
        _layer2_body,
        grid=(grid,),
        in_specs=[
            pl.BlockSpec((2, _RB, d_hid), lambda i: (0, i, 0)),
            pl.BlockSpec((_RB, 1), lambda i: (i, 0)),
            pl.BlockSpec((1, d_hid), lambda i: (0, 0)),
            pl.BlockSpec((d_hid, d_hid), lambda i: (0, 0)),
        ],
        out_specs=pl.BlockSpec((_RB, d_hid), lambda i: (i, 0)),
        out_shape=jax.ShapeDtypeStruct((n, d_hid), jnp.float32),
    )(p1, dis, b1.reshape(1, -1), W2)

    p2 = agg(g2, src_pad, dst_pad)

    out = pl.pallas_call(
        _final_body,
        grid=(grid,),
        in_specs=[
            pl.BlockSpec((2, _RB, d_hid), lambda i: (0, i, 0)),
            pl.BlockSpec((_RB, 1), lambda i: (i, 0)),
            pl.BlockSpec((1, d_hid), lambda i: (0, 0)),
            pl.BlockSpec((_RB, 1), lambda i: (i, 0)),
            pl.BlockSpec((d_hid, 1), lambda i: (0, 0)),
            pl.BlockSpec((1, 1), lambda i: (0, 0)),
        ],
        out_specs=pl.BlockSpec((g_graphs, 1), lambda i: (0, 0)),
        out_shape=jax.ShapeDtypeStruct((g_graphs, 1), jnp.float32),
        scratch_shapes=[
            pltpu.VMEM((g_graphs, d_hid), jnp.float32),
            pltpu.VMEM((g_graphs, 1), jnp.float32),
        ],
    )(p2, dis, b2.reshape(1, -1), batch.reshape(-1, 1), Wl,
      bl.reshape(1, 1))

    return out.reshape(-1)


# speedup vs baseline: 36.2224x; 5.3640x over previous
import functools

import jax
import jax.numpy as jnp
from jax import lax
from jax.experimental import pallas as pl
from jax.experimental.pallas import tpu as pltpu
from jax.experimental.pallas import tpu_sc as plsc

_NC = 2
_NS = 16
_NW = _NC * _NS
_BLK = 128
_NBUF = 4
_RB = 400


def _cdiv(a, b):
    return (a + b - 1) // b


def _make_sc_deg(R, RPT, NBT):
    mesh = plsc.VectorSubcoreMesh(core_axis_name="c", subcore_axis_name="s")

    @functools.partial(
        pl.kernel,
        out_type=jax.ShapeDtypeStruct((_NC, R, 16), jnp.float32),
        mesh=mesh,
        scratch_types=[
            pltpu.VMEM((NBT, _BLK), jnp.int32),
            pltpu.VMEM((_BLK, 16), jnp.float32),
            pltpu.VMEM((RPT, 16), jnp.float32),
            pltpu.VMEM_SHARED((R, 16), jnp.float32),
        ],
        compiler_params=pltpu.CompilerParams(use_tc_tiling_on_sc=False),
    )
    def deg_kernel(dst_hbm, out_hbm, didx, ones_b, bounce, acc):
        c = lax.axis_index("c")
        s = lax.axis_index("s")
        wid = c * _NS + s
        pltpu.sync_copy(dst_hbm.at[pl.ds(wid * NBT, NBT)], didx)

        ov = jnp.ones((16,), jnp.float32)
        zv = jnp.zeros((16,), jnp.float32)

        def fill(r, _):
            ones_b[r, :] = ov
            return 0

        lax.fori_loop(0, _BLK, fill, 0)

        def zero(r, _):
            bounce[r, :] = zv
            return 0

        lax.fori_loop(0, RPT, zero, 0)
        pltpu.sync_copy(bounce, acc.at[pl.ds(s * RPT, RPT)])
        plsc.subcore_barrier()

        def body(i, _):
            pltpu.sync_copy(ones_b, acc.at[didx.at[i]], add=True)
            return 0

        lax.fori_loop(0, NBT, body, 0)
        plsc.subcore_barrier()

        pltpu.sync_copy(acc.at[pl.ds(s * RPT, RPT)], bounce)
        pltpu.sync_copy(bounce, out_hbm.at[c].at[pl.ds(s * RPT, RPT)])

    return deg_kernel


def _make_sc_agg(R, RPT, NBT, D):
    mesh = plsc.VectorSubcoreMesh(core_axis_name="c", subcore_axis_name="s")

    @functools.partial(
        pl.kernel,
        out_type=jax.ShapeDtypeStruct((_NC, R, D), jnp.float32),
        mesh=mesh,
        scratch_types=[
            pltpu.VMEM((NBT, _BLK), jnp.int32),
            pltpu.VMEM((NBT, _BLK), jnp.int32),
            pltpu.VMEM((_NBUF, _BLK, D), jnp.float32),
            pltpu.VMEM_SHARED((R, D), jnp.float32),
            pltpu.SemaphoreType.DMA,
        ],
        compiler_params=pltpu.CompilerParams(use_tc_tiling_on_sc=False),
    )
    def agg_kernel(g_hbm, src_hbm, dst_hbm, zeros_hbm, out_hbm, sidx, didx,
                   rows, acc, sem):
        c = lax.axis_index("c")
        s = lax.axis_index("s")
        wid = c * _NS + s
        pltpu.sync_copy(src_hbm.at[pl.ds(wid * NBT, NBT)], sidx)
        pltpu.sync_copy(dst_hbm.at[pl.ds(wid * NBT, NBT)], didx)
        pltpu.sync_copy(zeros_hbm.at[pl.ds(s * RPT, RPT)],
                        acc.at[pl.ds(s * RPT, RPT)])
        plsc.subcore_barrier()

        for b in range(_NBUF):
            pltpu.async_copy(g_hbm.at[sidx.at[b]], rows.at[b], sem)

        def body(i, _):
            slot = lax.rem(i, _NBUF)
            pltpu.make_async_copy(g_hbm.at[sidx.at[i]], rows.at[slot],
                                  sem).wait()
            pltpu.sync_copy(rows.at[slot], acc.at[didx.at[i]], add=True)
            nxt = i + _NBUF

            @pl.when(nxt < NBT)
            def _():
                pltpu.async_copy(g_hbm.at[sidx.at[nxt]],
                                 rows.at[lax.rem(nxt, _NBUF)], sem)

            return 0

        lax.fori_loop(0, NBT, body, 0)
        plsc.subcore_barrier()

        pltpu.sync_copy(acc.at[pl.ds(s * RPT, RPT)],
                        out_hbm.at[c].at[pl.ds(s * RPT, RPT)])

    return agg_kernel


def _mm_body(x_ref, w_ref, o_ref):
    o_ref[...] = jnp.dot(x_ref[...], w_ref[...],
                         preferred_element_type=jnp.float32)


def _scale_body(dp_ref, h0_ref, g_ref, dis_ref):
    deg = dp_ref[0, :, 0:1] + dp_ref[1, :, 0:1]
    dis = lax.rsqrt(deg)
    dis_ref[...] = dis
    g_ref[...] = h0_ref[...] * dis


def _layer2_body(p_ref, dis_ref, b1_ref, w2_ref, g2_ref):
    dis = dis_ref[...]
    h1 = jnp.maximum((p_ref[0] + p_ref[1]) * dis + b1_ref[...], 0.0)
    g2_ref[...] = jnp.dot(h1, w2_ref[...],
                          preferred_element_type=jnp.float32) * dis


def _final_body(p_ref, dis_ref, b2_ref, batch_ref, wl_ref, bl_ref, o_ref,
                acc, cnt):
    i = pl.program_id(0)

    @pl.when(i == 0)
    def _():
        acc[...] = jnp.zeros_like(acc)
        cnt[...] = jnp.zeros_like(cnt)

    h2 = jnp.maximum((p_ref[0] + p_ref[1]) * dis_ref[...] + b2_ref[...], 0.0)
    gids = batch_ref[...]
    oh = (gids == lax.broadcasted_iota(jnp.int32, (1, 64), 1))
    oh = oh.astype(jnp.float32)
    acc[...] += lax.dot_general(oh, h2, (((0,), (0,)), ((), ())),
                                preferred_element_type=jnp.float32)
    cnt[...] += lax.dot_general(oh, jnp.ones((oh.shape[0], 1), jnp.float32),
                                (((0,), (0,)), ((), ())),
                                preferred_element_type=jnp.float32)

    @pl.when(i == pl.num_programs(0) - 1)
    def _():
        pooled = jnp.dot(acc[...], wl_ref[...],
                         preferred_element_type=jnp.float32)
        o_ref[...] = pooled / jnp.maximum(cnt[...], 1.0) + bl_ref[...]


def kernel(x, edge_index, batch, W1, b1, W2, b2, Wl, bl):
    n, d_in = x.shape
    d_hid = W1.shape[1]
    e = edge_index.shape[1]
    g_graphs = 64

    e_tot = e + n
    nbt = _cdiv(e_tot, _NW * _BLK * 8) * 8
    e_pad = nbt * _NW * _BLK
    rpt = _cdiv(n + 1, _NS * 8) * 8
    R = rpt * _NS

    loop = jnp.arange(n, dtype=jnp.int32)
    pad = e_pad - e_tot
    pad_ar = jnp.arange(pad, dtype=jnp.int32)
    src_fill = pad_ar % jnp.int32(n)
    dst_fill = n + pad_ar % jnp.int32(R - n)
    src_pad = jnp.concatenate(
        [edge_index[0], loop, src_fill]).reshape(-1, _BLK)
    dst_pad = jnp.concatenate(
        [edge_index[1], loop, dst_fill]).reshape(-1, _BLK)

    deg_parts = _make_sc_deg(R, rpt, nbt)(dst_pad)

    grid = n // _RB
    h0 = pl.pallas_call(
        _mm_body,
        grid=(grid,),
        in_specs=[
            pl.BlockSpec((_RB, d_in), lambda i: (i, 0)),
            pl.BlockSpec((d_in, d_hid), lambda i: (0, 0)),
        ],
        out_specs=pl.BlockSpec((_RB, d_hid), lambda i: (i, 0)),
        out_shape=jax.ShapeDtypeStruct((n, d_hid), jnp.float32),
    )(x, W1)

    g1, dis = pl.pallas_call(
        _scale_body,
        grid=(grid,),
        in_specs=[
            pl.BlockSpec((2, _RB, 16), lambda i: (0, i, 0)),
            pl.BlockSpec((_RB, d_hid), lambda i: (i, 0)),
        ],
        out_specs=[
            pl.BlockSpec((_RB, d_hid), lambda i: (i, 0)),
            pl.BlockSpec((_RB, 1), lambda i: (i, 0)),
        ],
        out_shape=[
            jax.ShapeDtypeStruct((n, d_hid), jnp.float32),
            jax.ShapeDtypeStruct((n, 1), jnp.float32),
        ],
    )(deg_parts, h0)

    zeros_acc = jnp.zeros((R, d_hid), jnp.float32)
    agg = _make_sc_agg(R, rpt, nbt, d_hid)
    p1 = agg(g1, src_pad, dst_pad, zeros_acc)

    g2 = pl.pallas_call(
        _layer2_body,
        grid=(grid,),
        in_specs=[
            pl.BlockSpec((2, _RB, d_hid), lambda i: (0, i, 0)),
            pl.BlockSpec((_RB, 1), lambda i: (i, 0)),
            pl.BlockSpec((1, d_hid), lambda i: (0, 0)),
            pl.BlockSpec((d_hid, d_hid), lambda i: (0, 0)),
        ],
        out_specs=pl.BlockSpec((_RB, d_hid), lambda i: (i, 0)),
        out_shape=jax.ShapeDtypeStruct((n, d_hid), jnp.float32),
    )(p1, dis, b1.reshape(1, -1), W2)

    p2 = agg(g2, src_pad, dst_pad, zeros_acc)

    out = pl.pallas_call(
        _final_body,
        grid=(grid,),
        in_specs=[
            pl.BlockSpec((2, _RB, d_hid), lambda i: (0, i, 0)),
            pl.BlockSpec((_RB, 1), lambda i: (i, 0)),
            pl.BlockSpec((1, d_hid), lambda i: (0, 0)),
            pl.BlockSpec((_RB, 1), lambda i: (i, 0)),
            pl.BlockSpec((d_hid, 1), lambda i: (0, 0)),
            pl.BlockSpec((1, 1), lambda i: (0, 0)),
        ],
        out_specs=pl.BlockSpec((g_graphs, 1), lambda i: (0, 0)),
        out_shape=jax.ShapeDtypeStruct((g_graphs, 1), jnp.float32),
        scratch_shapes=[
            pltpu.VMEM((g_graphs, d_hid), jnp.float32),
            pltpu.VMEM((g_graphs, 1), jnp.float32),
        ],
    )(p2, dis, b2.reshape(1, -1), batch.reshape(-1, 1), Wl,
      bl.reshape(1, 1))

    return out.reshape(-1)


# speedup vs baseline: 45.0226x; 1.2430x over previous
import functools

import jax
import jax.numpy as jnp
from jax import lax
from jax.experimental import pallas as pl
from jax.experimental.pallas import tpu as pltpu
from jax.experimental.pallas import tpu_sc as plsc

_NC = 2
_NS = 16
_NW = _NC * _NS
_BLK = 128
_NBUF = 6
_RB = 2000


def _tile_blocks(w, NB):
    nfull = NB // _NW
    rem = NB % _NW
    base = nfull * w + jnp.minimum(w, rem)
    cnt = nfull + jnp.where(w < rem, 1, 0)
    return base, cnt


def _make_sc_deg(n, RPT, NB, NBT):
    mesh = plsc.VectorSubcoreMesh(core_axis_name="c", subcore_axis_name="s")

    @functools.partial(
        pl.kernel,
        out_type=jax.ShapeDtypeStruct((_NC, n, 16), jnp.float32),
        mesh=mesh,
        scratch_types=[
            pltpu.VMEM((NBT, _BLK), jnp.int32),
            pltpu.VMEM((_BLK, 16), jnp.float32),
            pltpu.VMEM_SHARED((n, 16), jnp.float32),
        ],
        compiler_params=pltpu.CompilerParams(use_tc_tiling_on_sc=False),
    )
    def deg_kernel(dst_hbm, zeros_hbm, out_hbm, didx, ones_b, acc):
        c = lax.axis_index("c")
        s = lax.axis_index("s")
        w = c * _NS + s
        base, cnt = _tile_blocks(w, NB)
        pltpu.sync_copy(dst_hbm.at[pl.ds(base, NBT - 1)],
                        didx.at[pl.ds(0, NBT - 1)])

        @pl.when(cnt == NBT)
        def _():
            pltpu.sync_copy(dst_hbm.at[pl.ds(base + NBT - 1, 1)],
                            didx.at[pl.ds(NBT - 1, 1)])

        ov = jnp.ones((16,), jnp.float32)

        def fill(r, _):
            ones_b[r, :] = ov
            return 0

        lax.fori_loop(0, _BLK, fill, 0)
        pltpu.sync_copy(zeros_hbm.at[pl.ds(s * RPT, RPT)],
                        acc.at[pl.ds(s * RPT, RPT)])
        plsc.subcore_barrier()

        def body(i, _):
            pltpu.sync_copy(ones_b, acc.at[didx.at[i]], add=True)
            return 0

        lax.fori_loop(0, cnt, body, 0)
        plsc.subcore_barrier()

        pltpu.sync_copy(acc.at[pl.ds(s * RPT, RPT)],
                        out_hbm.at[c].at[pl.ds(s * RPT, RPT)])

    return deg_kernel


def _make_sc_agg(n, RPT, NB, NBT, D):
    mesh = plsc.VectorSubcoreMesh(core_axis_name="c", subcore_axis_name="s")

    @functools.partial(
        pl.kernel,
        out_type=jax.ShapeDtypeStruct((_NC, n, D), jnp.float32),
        mesh=mesh,
        scratch_types=[
            pltpu.VMEM((NBT, _BLK), jnp.int32),
            pltpu.VMEM((NBT, _BLK), jnp.int32),
            pltpu.VMEM((_NBUF, _BLK, D), jnp.float32),
            pltpu.VMEM_SHARED((n, D), jnp.float32),
            pltpu.SemaphoreType.DMA,
            pltpu.SemaphoreType.DMA,
        ],
        compiler_params=pltpu.CompilerParams(use_tc_tiling_on_sc=False),
    )
    def agg_kernel(g_hbm, src_hbm, dst_hbm, zeros_hbm, out_hbm, sidx, didx,
                   rows, acc, sem_g, sem_s):
        c = lax.axis_index("c")
        s = lax.axis_index("s")
        w = c * _NS + s
        base, cnt = _tile_blocks(w, NB)
        pltpu.sync_copy(src_hbm.at[pl.ds(base, NBT - 1)],
                        sidx.at[pl.ds(0, NBT - 1)])
        pltpu.sync_copy(dst_hbm.at[pl.ds(base, NBT - 1)],
                        didx.at[pl.ds(0, NBT - 1)])

        @pl.when(cnt == NBT)
        def _():
            pltpu.sync_copy(src_hbm.at[pl.ds(base + NBT - 1, 1)],
                            sidx.at[pl.ds(NBT - 1, 1)])
            pltpu.sync_copy(dst_hbm.at[pl.ds(base + NBT - 1, 1)],
                            didx.at[pl.ds(NBT - 1, 1)])

        pltpu.sync_copy(zeros_hbm.at[pl.ds(s * RPT, RPT)],
                        acc.at[pl.ds(s * RPT, RPT)])
        plsc.subcore_barrier()

        for b in range(_NBUF):
            pltpu.async_copy(g_hbm.at[sidx.at[b]], rows.at[b], sem_g)

        def body(i, _):
            @pl.when(i >= 1)
            def _():
                pltpu.make_async_copy(rows.at[lax.rem(i - 1, _NBUF)],
                                      acc.at[didx.at[i - 1]], sem_s).wait()
                nxt = i + _NBUF - 1

                @pl.when(nxt < cnt)
                def _():
                    pltpu.async_copy(g_hbm.at[sidx.at[nxt]],
                                     rows.at[lax.rem(nxt, _NBUF)], sem_g)

            slot = lax.rem(i, _NBUF)
            pltpu.make_async_copy(g_hbm.at[sidx.at[i]], rows.at[slot],
                                  sem_g).wait()
            pltpu.async_copy(rows.at[slot], acc.at[didx.at[i]], sem_s,
                             add=True)
            return 0

        lax.fori_loop(0, cnt, body, 0)
        pltpu.make_async_copy(rows.at[lax.rem(cnt - 1, _NBUF)],
                              acc.at[didx.at[cnt - 1]], sem_s).wait()
        plsc.subcore_barrier()

        pltpu.sync_copy(acc.at[pl.ds(s * RPT, RPT)],
                        out_hbm.at[c].at[pl.ds(s * RPT, RPT)])

    return agg_kernel


def _mm_body(x_ref, w_ref, o_ref):
    o_ref[...] = jnp.dot(x_ref[...], w_ref[...],
                         preferred_element_type=jnp.float32)


def _scale_body(dp_ref, h0_ref, g_ref, dis_ref):
    deg = dp_ref[0, :, 0:1] + dp_ref[1, :, 0:1] + 1.0
    dis = lax.rsqrt(deg)
    dis_ref[...] = dis
    g_ref[...] = h0_ref[...] * dis


def _layer2_body(p_ref, g1_ref, dis_ref, b1_ref, w2_ref, g2_ref):
    dis = dis_ref[...]
    agg = p_ref[0] + p_ref[1] + g1_ref[...]
    h1 = jnp.maximum(agg * dis + b1_ref[...], 0.0)
    g2_ref[...] = jnp.dot(h1, w2_ref[...],
                          preferred_element_type=jnp.float32) * dis


def _final_body(p_ref, g2_ref, dis_ref, b2_ref, batch_ref, wl_ref, bl_ref,
                o_ref, acc, cnt):
    i = pl.program_id(0)

    @pl.when(i == 0)
    def _():
        acc[...] = jnp.zeros_like(acc)
        cnt[...] = jnp.zeros_like(cnt)

    agg = p_ref[0] + p_ref[1] + g2_ref[...]
    h2 = jnp.maximum(agg * dis_ref[...] + b2_ref[...], 0.0)
    gids = batch_ref[...]
    oh = (gids == lax.broadcasted_iota(jnp.int32, (1, 64), 1))
    oh = oh.astype(jnp.float32)
    acc[...] += lax.dot_general(oh, h2, (((0,), (0,)), ((), ())),
                                preferred_element_type=jnp.float32)
    cnt[...] += lax.dot_general(oh, jnp.ones((oh.shape[0], 1), jnp.float32),
                                (((0,), (0,)), ((), ())),
                                preferred_element_type=jnp.float32)

    @pl.when(i == pl.num_programs(0) - 1)
    def _():
        pooled = jnp.dot(acc[...], wl_ref[...],
                         preferred_element_type=jnp.float32)
        o_ref[...] = pooled / jnp.maximum(cnt[...], 1.0) + bl_ref[...]


def kernel(x, edge_index, batch, W1, b1, W2, b2, Wl, bl):
    n, d_in = x.shape
    d_hid = W1.shape[1]
    e = edge_index.shape[1]
    g_graphs = 64

    NB = e // _BLK
    NBT = NB // _NW + 1
    rpt = n // _NS

    src2d = edge_index[0].reshape(NB, _BLK)
    dst2d = edge_index[1].reshape(NB, _BLK)

    zeros16 = jnp.zeros((n, 16), jnp.float32)
    zeros_acc = jnp.zeros((n, d_hid), jnp.float32)

    deg_parts = _make_sc_deg(n, rpt, NB, NBT)(dst2d, zeros16)

    grid = n // _RB
    h0 = pl.pallas_call(
        _mm_body,
        grid=(grid,),
        in_specs=[
            pl.BlockSpec((_RB, d_in), lambda i: (i, 0)),
            pl.BlockSpec((d_in, d_hid), lambda i: (0, 0)),
        ],
        out_specs=pl.BlockSpec((_RB, d_hid), lambda i: (i, 0)),
        out_shape=jax.ShapeDtypeStruct((n, d_hid), jnp.float32),
    )(x, W1)

    g1, dis = pl.pallas_call(
        _scale_body,
        grid=(grid,),
        in_specs=[
            pl.BlockSpec((2, _RB, 16), lambda i: (0, i, 0)),
            pl.BlockSpec((_RB, d_hid), lambda i: (i, 0)),
        ],
        out_specs=[
            pl.BlockSpec((_RB, d_hid), lambda i: (i, 0)),
            pl.BlockSpec((_RB, 1), lambda i: (i, 0)),
        ],
        out_shape=[
            jax.ShapeDtypeStruct((n, d_hid), jnp.float32),
            jax.ShapeDtypeStruct((n, 1), jnp.float32),
        ],
    )(deg_parts, h0)

    agg = _make_sc_agg(n, rpt, NB, NBT, d_hid)
    p1 = agg(g1, src2d, dst2d, zeros_acc)

    g2 = pl.pallas_call(
        _layer2_body,
        grid=(grid,),
        in_specs=[
            pl.BlockSpec((2, _RB, d_hid), lambda i: (0, i, 0)),
            pl.BlockSpec((_RB, d_hid), lambda i: (i, 0)),
            pl.BlockSpec((_RB, 1), lambda i: (i, 0)),
            pl.BlockSpec((1, d_hid), lambda i: (0, 0)),
            pl.BlockSpec((d_hid, d_hid), lambda i: (0, 0)),
        ],
        out_specs=pl.BlockSpec((_RB, d_hid), lambda i: (i, 0)),
        out_shape=jax.ShapeDtypeStruct((n, d_hid), jnp.float32),
    )(p1, g1, dis, b1.reshape(1, -1), W2)

    p2 = agg(g2, src2d, dst2d, zeros_acc)

    out = pl.pallas_call(
        _final_body,
        grid=(grid,),
        in_specs=[
            pl.BlockSpec((2, _RB, d_hid), lambda i: (0, i, 0)),
            pl.BlockSpec((_RB, d_hid), lambda i: (i, 0)),
            pl.BlockSpec((_RB, 1), lambda i: (i, 0)),
            pl.BlockSpec((1, d_hid), lambda i: (0, 0)),
            pl.BlockSpec((_RB, 1), lambda i: (i, 0)),
            pl.BlockSpec((d_hid, 1), lambda i: (0, 0)),
            pl.BlockSpec((1, 1), lambda i: (0, 0)),
        ],
        out_specs=pl.BlockSpec((g_graphs, 1), lambda i: (0, 0)),
        out_shape=jax.ShapeDtypeStruct((g_graphs, 1), jnp.float32),
        scratch_shapes=[
            pltpu.VMEM((g_graphs, d_hid), jnp.float32),
            pltpu.VMEM((g_graphs, 1), jnp.float32),
        ],
    )(p2, g2, dis, b2.reshape(1, -1), batch.reshape(-1, 1), Wl,
      bl.reshape(1, 1))

    return out.reshape(-1)


# speedup vs baseline: 49.3216x; 1.0955x over previous
import functools

import jax
import jax.numpy as jnp
from jax import lax
from jax.experimental import pallas as pl
from jax.experimental.pallas import tpu as pltpu
from jax.experimental.pallas import tpu_sc as plsc

_NC = 2
_NS = 16
_NW = _NC * _NS
_BLK = 128
_NBUF = 6
_RB = 1000


def _tile_blocks(w, NB):
    nfull = NB // _NW
    rem = NB % _NW
    base = nfull * w + jnp.minimum(w, rem)
    cnt = nfull + jnp.where(w < rem, 1, 0)
    return base, cnt


def _make_sc_deg(n, RPT, NB, NBT):
    mesh = plsc.VectorSubcoreMesh(core_axis_name="c", subcore_axis_name="s")

    @functools.partial(
        pl.kernel,
        out_type=jax.ShapeDtypeStruct((_NC, n, 16), jnp.float32),
        mesh=mesh,
        scratch_types=[
            pltpu.VMEM((NBT, _BLK), jnp.int32),
            pltpu.VMEM((_BLK, 16), jnp.float32),
            pltpu.VMEM_SHARED((n, 16), jnp.float32),
        ],
        compiler_params=pltpu.CompilerParams(use_tc_tiling_on_sc=False),
    )
    def deg_kernel(dst_hbm, zeros_hbm, out_hbm, didx, ones_b, acc):
        c = lax.axis_index("c")
        s = lax.axis_index("s")
        w = c * _NS + s
        base, cnt = _tile_blocks(w, NB)
        pltpu.sync_copy(dst_hbm.at[pl.ds(base, NBT - 1)],
                        didx.at[pl.ds(0, NBT - 1)])

        @pl.when(cnt == NBT)
        def _():
            pltpu.sync_copy(dst_hbm.at[pl.ds(base + NBT - 1, 1)],
                            didx.at[pl.ds(NBT - 1, 1)])

        ov = jnp.ones((16,), jnp.float32)

        def fill(r, _):
            ones_b[r, :] = ov
            return 0

        lax.fori_loop(0, _BLK, fill, 0)
        pltpu.sync_copy(zeros_hbm.at[pl.ds(s * RPT, RPT)],
                        acc.at[pl.ds(s * RPT, RPT)])
        plsc.subcore_barrier()

        def body(i, _):
            pltpu.sync_copy(ones_b, acc.at[didx.at[i]], add=True)
            return 0

        lax.fori_loop(0, cnt, body, 0)
        plsc.subcore_barrier()

        pltpu.sync_copy(acc.at[pl.ds(s * RPT, RPT)],
                        out_hbm.at[c].at[pl.ds(s * RPT, RPT)])

    return deg_kernel


def _make_sc_agg(n, RPT, NB, NBT, D):
    mesh = plsc.VectorSubcoreMesh(core_axis_name="c", subcore_axis_name="s")

    @functools.partial(
        pl.kernel,
        out_type=jax.ShapeDtypeStruct((_NC, n, D), jnp.float32),
        mesh=mesh,
        scratch_types=[
            pltpu.VMEM((NBT, _BLK), jnp.int32),
            pltpu.VMEM((NBT, _BLK), jnp.int32),
            pltpu.VMEM((_NBUF, _BLK, D), jnp.float32),
            pltpu.VMEM_SHARED((n, D), jnp.float32),
            pltpu.SemaphoreType.DMA,
            pltpu.SemaphoreType.DMA,
        ],
        compiler_params=pltpu.CompilerParams(use_tc_tiling_on_sc=False),
    )
    def agg_kernel(g_hbm, src_hbm, dst_hbm, zeros_hbm, out_hbm, sidx, didx,
                   rows, acc, sem_g, sem_s):
        c = lax.axis_index("c")
        s = lax.axis_index("s")
        w = c * _NS + s
        base, cnt = _tile_blocks(w, NB)
        pltpu.sync_copy(src_hbm.at[pl.ds(base, NBT - 1)],
                        sidx.at[pl.ds(0, NBT - 1)])
        pltpu.sync_copy(dst_hbm.at[pl.ds(base, NBT - 1)],
                        didx.at[pl.ds(0, NBT - 1)])

        @pl.when(cnt == NBT)
        def _():
            pltpu.sync_copy(src_hbm.at[pl.ds(base + NBT - 1, 1)],
                            sidx.at[pl.ds(NBT - 1, 1)])
            pltpu.sync_copy(dst_hbm.at[pl.ds(base + NBT - 1, 1)],
                            didx.at[pl.ds(NBT - 1, 1)])

        pltpu.sync_copy(zeros_hbm.at[pl.ds(s * RPT, RPT)],
                        acc.at[pl.ds(s * RPT, RPT)])
        plsc.subcore_barrier()

        for b in range(_NBUF):
            pltpu.async_copy(g_hbm.at[sidx.at[b]], rows.at[b], sem_g)

        def body(i, _):
            @pl.when(i >= 1)
            def _():
                pltpu.make_async_copy(rows.at[lax.rem(i - 1, _NBUF)],
                                      acc.at[didx.at[i - 1]], sem_s).wait()
                nxt = i + _NBUF - 1

                @pl.when(nxt < cnt)
                def _():
                    pltpu.async_copy(g_hbm.at[sidx.at[nxt]],
                                     rows.at[lax.rem(nxt, _NBUF)], sem_g)

            slot = lax.rem(i, _NBUF)
            pltpu.make_async_copy(g_hbm.at[sidx.at[i]], rows.at[slot],
                                  sem_g).wait()
            pltpu.async_copy(rows.at[slot], acc.at[didx.at[i]], sem_s,
                             add=True)
            return 0

        lax.fori_loop(0, cnt, body, 0)
        pltpu.make_async_copy(rows.at[lax.rem(cnt - 1, _NBUF)],
                              acc.at[didx.at[cnt - 1]], sem_s).wait()
        plsc.subcore_barrier()

        pltpu.sync_copy(acc.at[pl.ds(s * RPT, RPT)],
                        out_hbm.at[c].at[pl.ds(s * RPT, RPT)])

    return agg_kernel


def _mm_body(x_ref, w_ref, o_ref):
    o_ref[...] = jnp.dot(x_ref[...], w_ref[...],
                         preferred_element_type=jnp.float32)


def _scale_body(dpl_ref, dpr_ref, h0l_ref, h0r_ref, g_ref, dis_ref):
    degl = dpl_ref[0, :, 0:1] + dpl_ref[1, :, 0:1] + 1.0
    degr = dpr_ref[0, :, 0:1] + dpr_ref[1, :, 0:1] + 1.0
    disl = lax.rsqrt(degl)
    disr = lax.rsqrt(degr)
    b = disl.shape[0]
    dis_ref[...] = jnp.concatenate(
        [jnp.broadcast_to(disl, (b, 64)), jnp.broadcast_to(disr, (b, 64))],
        axis=1)
    g_ref[...] = jnp.concatenate(
        [h0l_ref[...] * disl, h0r_ref[...] * disr], axis=1)


def _layer2_body(p_ref, g1_ref, dis_ref, b1_ref, w2_ref, g2_ref):
    dis = dis_ref[...]
    agg = p_ref[0] + p_ref[1] + g1_ref[...]
    h1 = jnp.maximum(agg * dis + b1_ref[...], 0.0)
    g2_ref[...] = jnp.dot(h1, w2_ref[...],
                          preferred_element_type=jnp.float32) * dis


def _final_body(p_ref, g2_ref, dis_ref, b2_ref, bl_ref2, br_ref2, wl_ref,
                blb_ref, o_ref, acc, cnt):
    i = pl.program_id(0)

    @pl.when(i == 0)
    def _():
        acc[...] = jnp.zeros_like(acc)
        cnt[...] = jnp.zeros_like(cnt)

    agg = p_ref[0] + p_ref[1] + g2_ref[...]
    h2 = jnp.maximum(agg * dis_ref[...] + b2_ref[...], 0.0)
    h2l = h2[:, 0:64]
    h2r = h2[:, 64:128]
    iota = lax.broadcasted_iota(jnp.int32, (1, 64), 1)
    ohl = (bl_ref2[...] == iota).astype(jnp.float32)
    ohr = (br_ref2[...] == iota).astype(jnp.float32)
    acc[...] += (
        lax.dot_general(ohl, h2l, (((0,), (0,)), ((), ())),
                        preferred_element_type=jnp.float32)
        + lax.dot_general(ohr, h2r, (((0,), (0,)), ((), ())),
                          preferred_element_type=jnp.float32))
    ones = jnp.ones((ohl.shape[0], 1), jnp.float32)
    cnt[...] += lax.dot_general(ohl + ohr, ones, (((0,), (0,)), ((), ())),
                                preferred_element_type=jnp.float32)

    @pl.when(i == pl.num_programs(0) - 1)
    def _():
        pooled = jnp.dot(acc[...], wl_ref[...],
                         preferred_element_type=jnp.float32)
        o_ref[...] = pooled / jnp.maximum(cnt[...], 1.0) + blb_ref[...]


def kernel(x, edge_index, batch, W1, b1, W2, b2, Wl, bl):
    n, d_in = x.shape
    d_hid = W1.shape[1]
    e = edge_index.shape[1]
    g_graphs = 64
    nh = n // 2
    dp2 = 2 * d_hid

    NB = e // _BLK
    NBT = NB // _NW + 1
    rpt = n // _NS

    ei_t = (edge_index % nh) * 2 + edge_index // nh
    src_t = ei_t[0].reshape(NB, _BLK)
    dst_t = ei_t[1].reshape(NB, _BLK)
    dst_raw = edge_index[1].reshape(NB, _BLK)

    zeros16 = jnp.zeros((n, 16), jnp.float32)
    zeros_acc = jnp.zeros((n, d_hid), jnp.float32)

    deg_parts = _make_sc_deg(n, rpt, NB, NBT)(dst_raw, zeros16)

    grid = nh // _RB
    h0 = pl.pallas_call(
        _mm_body,
        grid=(2 * grid,),
        in_specs=[
            pl.BlockSpec((_RB, d_in), lambda i: (i, 0)),
            pl.BlockSpec((d_in, d_hid), lambda i: (0, 0)),
        ],
        out_specs=pl.BlockSpec((_RB, d_hid), lambda i: (i, 0)),
        out_shape=jax.ShapeDtypeStruct((n, d_hid), jnp.float32),
    )(x, W1)

    g1p, disp = pl.pallas_call(
        _scale_body,
        grid=(grid,),
        in_specs=[
            pl.BlockSpec((2, _RB, 16), lambda i: (0, i, 0)),
            pl.BlockSpec((2, _RB, 16), lambda i, g=grid: (0, i + g, 0)),
            pl.BlockSpec((_RB, d_hid), lambda i: (i, 0)),
            pl.BlockSpec((_RB, d_hid), lambda i, g=grid: (i + g, 0)),
        ],
        out_specs=[
            pl.BlockSpec((_RB, dp2), lambda i: (i, 0)),
            pl.BlockSpec((_RB, dp2), lambda i: (i, 0)),
        ],
        out_shape=[
            jax.ShapeDtypeStruct((nh, dp2), jnp.float32),
            jax.ShapeDtypeStruct((nh, dp2), jnp.float32),
        ],
    )(deg_parts, deg_parts, h0, h0)

    agg = _make_sc_agg(n, rpt, NB, NBT, d_hid)
    p1 = agg(g1p.reshape(n, d_hid), src_t, dst_t, zeros_acc)
    p1p = p1.reshape(_NC, nh, dp2)

    w2blk = jnp.zeros((dp2, dp2), jnp.float32)
    w2blk = w2blk.at[:d_hid, :d_hid].set(W2).at[d_hid:, d_hid:].set(W2)
    b1p = jnp.tile(b1, 2).reshape(1, dp2)
    b2p = jnp.tile(b2, 2).reshape(1, dp2)

    g2p = pl.pallas_call(
        _layer2_body,
        grid=(grid,),
        in_specs=[
            pl.BlockSpec((2, _RB, dp2), lambda i: (0, i, 0)),
            pl.BlockSpec((_RB, dp2), lambda i: (i, 0)),
            pl.BlockSpec((_RB, dp2), lambda i: (i, 0)),
            pl.BlockSpec((1, dp2), lambda i: (0, 0)),
            pl.BlockSpec((dp2, dp2), lambda i: (0, 0)),
        ],
        out_specs=pl.BlockSpec((_RB, dp2), lambda i: (i, 0)),
        out_shape=jax.ShapeDtypeStruct((nh, dp2), jnp.float32),
    )(p1p, g1p, disp, b1p, w2blk)

    p2 = agg(g2p.reshape(n, d_hid), src_t, dst_t, zeros_acc)
    p2p = p2.reshape(_NC, nh, dp2)

    batch2 = batch.reshape(-1, 1)
    out = pl.pallas_call(
        _final_body,
        grid=(grid,),
        in_specs=[
            pl.BlockSpec((2, _RB, dp2), lambda i: (0, i, 0)),
            pl.BlockSpec((_RB, dp2), lambda i: (i, 0)),
            pl.BlockSpec((_RB, dp2), lambda i: (i, 0)),
            pl.BlockSpec((1, dp2), lambda i: (0, 0)),
            pl.BlockSpec((_RB, 1), lambda i: (i, 0)),
            pl.BlockSpec((_RB, 1), lambda i, g=grid: (i + g, 0)),
            pl.BlockSpec((d_hid, 1), lambda i: (0, 0)),
            pl.BlockSpec((1, 1), lambda i: (0, 0)),
        ],
        out_specs=pl.BlockSpec((g_graphs, 1), lambda i: (0, 0)),
        out_shape=jax.ShapeDtypeStruct((g_graphs, 1), jnp.float32),
        scratch_shapes=[
            pltpu.VMEM((g_graphs, d_hid), jnp.float32),
            pltpu.VMEM((g_graphs, 1), jnp.float32),
        ],
    )(p2p, g2p, disp, b2p, batch2, batch2, Wl, bl.reshape(1, 1))

    return out.reshape(-1)


# speedup vs baseline: 52.6310x; 1.0671x over previous
import functools

import jax
import jax.numpy as jnp
from jax import lax
from jax.experimental import pallas as pl
from jax.experimental.pallas import tpu as pltpu
from jax.experimental.pallas import tpu_sc as plsc

_NC = 2
_NS = 16
_NW = _NC * _NS
_BLK = 128
_NBUF = 8
_LAG = 2
_RB = 1000


def _tile_blocks(w, NB):
    nfull = NB // _NW
    rem = NB % _NW
    base = nfull * w + jnp.minimum(w, rem)
    cnt = nfull + jnp.where(w < rem, 1, 0)
    return base, cnt


def _make_sc_deg(n, RPT, NB, NBT, D):
    mesh = plsc.VectorSubcoreMesh(core_axis_name="c", subcore_axis_name="s")
    nh = n // 2

    @functools.partial(
        pl.kernel,
        out_type=[
            jax.ShapeDtypeStruct((_NC, n, D), jnp.float32),
            jax.ShapeDtypeStruct((NB, _BLK), jnp.int32),
            jax.ShapeDtypeStruct((NB, _BLK), jnp.int32),
        ],
        mesh=mesh,
        scratch_types=[
            pltpu.VMEM((NBT, _BLK), jnp.int32),
            pltpu.VMEM((NBT, _BLK), jnp.int32),
            pltpu.VMEM((NBT, _BLK), jnp.int32),
            pltpu.VMEM((_BLK, D), jnp.float32),
            pltpu.VMEM_SHARED((n, D), jnp.float32),
            pltpu.SemaphoreType.DMA,
        ],
        compiler_params=pltpu.CompilerParams(use_tc_tiling_on_sc=False),
    )
    def deg_kernel(ei2_hbm, zeros_hbm, out_hbm, srct_hbm, dstt_hbm,
                   sidx, didx, didx_t, ones_b, acc, sem_s):
        c = lax.axis_index("c")
        s = lax.axis_index("s")
        w = c * _NS + s
        base, cnt = _tile_blocks(w, NB)
        pltpu.sync_copy(ei2_hbm.at[pl.ds(base, NBT - 1)],
                        sidx.at[pl.ds(0, NBT - 1)])
        pltpu.sync_copy(ei2_hbm.at[pl.ds(NB + base, NBT - 1)],
                        didx.at[pl.ds(0, NBT - 1)])

        @pl.when(cnt == NBT)
        def _():
            pltpu.sync_copy(ei2_hbm.at[pl.ds(base + NBT - 1, 1)],
                            sidx.at[pl.ds(NBT - 1, 1)])
            pltpu.sync_copy(ei2_hbm.at[pl.ds(NB + base + NBT - 1, 1)],
                            didx.at[pl.ds(NBT - 1, 1)])

        ov = jnp.ones((16,), jnp.float32)

        def fill(r, _):
            for j in range(D // 16):
                ones_b[r, pl.ds(16 * j, 16)] = ov
            return 0

        lax.fori_loop(0, _BLK, fill, 0)

        def remap_dst(r, _):
            for j in range(_BLK // 16):
                v = didx[r, pl.ds(16 * j, 16)]
                t = v + v
                didx_t[r, pl.ds(16 * j, 16)] = jnp.where(
                    v >= nh, t - (n - 1), t)
            return 0

        lax.fori_loop(0, cnt, remap_dst, 0)

        pltpu.sync_copy(zeros_hbm.at[pl.ds(s * RPT, RPT)],
                        acc.at[pl.ds(s * RPT, RPT)])
        plsc.subcore_barrier()

        lag_d = 4

        def body(i, _):
            @pl.when(i >= lag_d)
            def _():
                pltpu.make_async_copy(ones_b, acc.at[didx_t.at[i - lag_d]],
                                      sem_s).wait()

            pltpu.async_copy(ones_b, acc.at[didx_t.at[i]], sem_s, add=True)
            for j in range(_BLK // 16):
                v = sidx[i, pl.ds(16 * j, 16)]
                t = v + v
                sidx[i, pl.ds(16 * j, 16)] = jnp.where(v >= nh, t - (n - 1), t)
            return 0

        lax.fori_loop(0, cnt, body, 0)

        def drain_deg(k, _):
            i = jnp.maximum(cnt - lag_d, 0) + k
            @pl.when(i < cnt)
            def _():
                pltpu.make_async_copy(ones_b, acc.at[didx_t.at[i]],
                                      sem_s).wait()
            return 0

        lax.fori_loop(0, lag_d, drain_deg, 0)

        pltpu.sync_copy(sidx.at[pl.ds(0, NBT - 1)],
                        srct_hbm.at[pl.ds(base, NBT - 1)])
        pltpu.sync_copy(didx_t.at[pl.ds(0, NBT - 1)],
                        dstt_hbm.at[pl.ds(base, NBT - 1)])

        @pl.when(cnt == NBT)
        def _():
            pltpu.sync_copy(sidx.at[pl.ds(NBT - 1, 1)],
                            srct_hbm.at[pl.ds(base + NBT - 1, 1)])
            pltpu.sync_copy(didx_t.at[pl.ds(NBT - 1, 1)],
                            dstt_hbm.at[pl.ds(base + NBT - 1, 1)])

        plsc.subcore_barrier()
        pltpu.sync_copy(acc.at[pl.ds(s * RPT, RPT)],
                        out_hbm.at[c].at[pl.ds(s * RPT, RPT)])

    return deg_kernel


def _make_sc_agg(n, RPT, NB, NBT, D):
    mesh = plsc.VectorSubcoreMesh(core_axis_name="c", subcore_axis_name="s")

    @functools.partial(
        pl.kernel,
        out_type=jax.ShapeDtypeStruct((_NC, n, D), jnp.float32),
        mesh=mesh,
        scratch_types=[
            pltpu.VMEM((NBT, _BLK), jnp.int32),
            pltpu.VMEM((NBT, _BLK), jnp.int32),
            pltpu.VMEM((_NBUF, _BLK, D), jnp.float32),
            pltpu.VMEM_SHARED((n, D), jnp.float32),
            pltpu.SemaphoreType.DMA,
            pltpu.SemaphoreType.DMA,
        ],
        compiler_params=pltpu.CompilerParams(use_tc_tiling_on_sc=False),
    )
    def agg_kernel(g_hbm, src_hbm, dst_hbm, zeros_hbm, out_hbm, sidx, didx,
                   rows, acc, sem_g, sem_s):
        c = lax.axis_index("c")
        s = lax.axis_index("s")
        w = c * _NS + s
        base, cnt = _tile_blocks(w, NB)
        pltpu.sync_copy(src_hbm.at[pl.ds(base, NBT - 1)],
                        sidx.at[pl.ds(0, NBT - 1)])
        pltpu.sync_copy(dst_hbm.at[pl.ds(base, NBT - 1)],
                        didx.at[pl.ds(0, NBT - 1)])

        @pl.when(cnt == NBT)
        def _():
            pltpu.sync_copy(src_hbm.at[pl.ds(base + NBT - 1, 1)],
                            sidx.at[pl.ds(NBT - 1, 1)])
            pltpu.sync_copy(dst_hbm.at[pl.ds(base + NBT - 1, 1)],
                            didx.at[pl.ds(NBT - 1, 1)])

        pltpu.sync_copy(zeros_hbm.at[pl.ds(s * RPT, RPT)],
                        acc.at[pl.ds(s * RPT, RPT)])
        plsc.subcore_barrier()

        for b in range(_NBUF):
            pltpu.async_copy(g_hbm.at[sidx.at[b]], rows.at[b], sem_g)

        def body(i, _):
            @pl.when(i >= _LAG)
            def _():
                pltpu.make_async_copy(rows.at[lax.rem(i - _LAG, _NBUF)],
                                      acc.at[didx.at[i - _LAG]], sem_s).wait()
                nxt = i + _NBUF - _LAG

                @pl.when(nxt < cnt)
                def _():
                    pltpu.async_copy(g_hbm.at[sidx.at[nxt]],
                                     rows.at[lax.rem(nxt, _NBUF)], sem_g)

            slot = lax.rem(i, _NBUF)
            pltpu.make_async_copy(g_hbm.at[sidx.at[i]], rows.at[slot],
                                  sem_g).wait()
            pltpu.async_copy(rows.at[slot], acc.at[didx.at[i]], sem_s,
                             add=True)
            return 0

        lax.fori_loop(0, cnt, body, 0)

        def drain(k, _):
            i = cnt - _LAG + k
            pltpu.make_async_copy(rows.at[lax.rem(i, _NBUF)],
                                  acc.at[didx.at[i]], sem_s).wait()
            return 0

        lax.fori_loop(0, _LAG, drain, 0)
        plsc.subcore_barrier()

        pltpu.sync_copy(acc.at[pl.ds(s * RPT, RPT)],
                        out_hbm.at[c].at[pl.ds(s * RPT, RPT)])

    return agg_kernel


def _mm_body(xa_ref, xb_ref, w_ref, o_ref):
    o_ref[...] = jnp.concatenate(
        [jnp.dot(xa_ref[...], w_ref[...], preferred_element_type=jnp.float32),
         jnp.dot(xb_ref[...], w_ref[...], preferred_element_type=jnp.float32)],
        axis=1)


def _scale_body(dp_ref, h0_ref, g_ref, dis_ref):
    deg = dp_ref[0] + dp_ref[1] + 1.0
    dis = lax.rsqrt(deg)
    dis_ref[...] = dis
    g_ref[...] = h0_ref[...] * dis


def _layer2_body(p_ref, g1_ref, dis_ref, b1_ref, w2_ref, g2_ref):
    dis = dis_ref[...]
    agg = p_ref[0] + p_ref[1] + g1_ref[...]
    h1 = jnp.maximum(agg * dis + b1_ref[...], 0.0)
    g2_ref[...] = jnp.dot(h1, w2_ref[...],
                          preferred_element_type=jnp.float32) * dis


def _final_body(p_ref, g2_ref, dis_ref, b2_ref, bl_ref2, br_ref2, wl_ref,
                blb_ref, o_ref, acc, cnt):
    i = pl.program_id(0)

    @pl.when(i == 0)
    def _():
        acc[...] = jnp.zeros_like(acc)
        cnt[...] = jnp.zeros_like(cnt)

    agg = p_ref[0] + p_ref[1] + g2_ref[...]
    h2 = jnp.maximum(agg * dis_ref[...] + b2_ref[...], 0.0)
    h2l = h2[:, 0:64]
    h2r = h2[:, 64:128]
    iota = lax.broadcasted_iota(jnp.int32, (1, 64), 1)
    ohl = (bl_ref2[...].astype(jnp.int32) == iota).astype(jnp.float32)
    ohr = (br_ref2[...].astype(jnp.int32) == iota).astype(jnp.float32)
    acc[...] += (
        lax.dot_general(ohl, h2l, (((0,), (0,)), ((), ())),
                        preferred_element_type=jnp.float32)
        + lax.dot_general(ohr, h2r, (((0,), (0,)), ((), ())),
                          preferred_element_type=jnp.float32))
    ones = jnp.ones((ohl.shape[0], 1), jnp.float32)
    cnt[...] += lax.dot_general(ohl + ohr, ones, (((0,), (0,)), ((), ())),
                                preferred_element_type=jnp.float32)

    @pl.when(i == pl.num_programs(0) - 1)
    def _():
        pooled = jnp.dot(acc[...], wl_ref[...],
                         preferred_element_type=jnp.float32)
        o_ref[...] = pooled / jnp.maximum(cnt[...], 1.0) + blb_ref[...]


def kernel(x, edge_index, batch, W1, b1, W2, b2, Wl, bl):
    n, d_in = x.shape
    d_hid = W1.shape[1]
    e = edge_index.shape[1]
    g_graphs = 64
    nh = n // 2
    dp2 = 2 * d_hid

    NB = e // _BLK
    NBT = NB // _NW + 1
    rpt = n // _NS

    ei2 = edge_index.reshape(2 * NB, _BLK)
    zeros_acc = jnp.zeros((n, d_hid), jnp.float32)

    deg_parts, src_t, dst_t = _make_sc_deg(n, rpt, NB, NBT, d_hid)(
        ei2, zeros_acc)
    dpv = deg_parts.reshape(_NC, nh, dp2)

    grid = nh // _RB
    h0p = pl.pallas_call(
        _mm_body,
        grid=(grid,),
        in_specs=[
            pl.BlockSpec((_RB, d_in), lambda i: (i, 0)),
            pl.BlockSpec((_RB, d_in), lambda i, g=grid: (i + g, 0)),
            pl.BlockSpec((d_in, d_hid), lambda i: (0, 0)),
        ],
        out_specs=pl.BlockSpec((_RB, dp2), lambda i: (i, 0)),
        out_shape=jax.ShapeDtypeStruct((nh, dp2), jnp.float32),
    )(x, x, W1)

    g1p, disp = pl.pallas_call(
        _scale_body,
        grid=(grid,),
        in_specs=[
            pl.BlockSpec((2, _RB, dp2), lambda i: (0, i, 0)),
            pl.BlockSpec((_RB, dp2), lambda i: (i, 0)),
        ],
        out_specs=[
            pl.BlockSpec((_RB, dp2), lambda i: (i, 0)),
            pl.BlockSpec((_RB, dp2), lambda i: (i, 0)),
        ],
        out_shape=[
            jax.ShapeDtypeStruct((nh, dp2), jnp.float32),
            jax.ShapeDtypeStruct((nh, dp2), jnp.float32),
        ],
    )(dpv, h0p)

    agg = _make_sc_agg(n, rpt, NB, NBT, d_hid)
    p1 = agg(g1p.reshape(n, d_hid), src_t, dst_t, zeros_acc)
    p1p = p1.reshape(_NC, nh, dp2)

    w2blk = jnp.zeros((dp2, dp2), jnp.float32)
    w2blk = w2blk.at[:d_hid, :d_hid].set(W2).at[d_hid:, d_hid:].set(W2)
    b1p = jnp.tile(b1, 2).reshape(1, dp2)
    b2p = jnp.tile(b2, 2).reshape(1, dp2)

    g2p = pl.pallas_call(
        _layer2_body,
        grid=(grid,),
        in_specs=[
            pl.BlockSpec((2, _RB, dp2), lambda i: (0, i, 0)),
            pl.BlockSpec((_RB, dp2), lambda i: (i, 0)),
            pl.BlockSpec((_RB, dp2), lambda i: (i, 0)),
            pl.BlockSpec((1, dp2), lambda i: (0, 0)),
            pl.BlockSpec((dp2, dp2), lambda i: (0, 0)),
        ],
        out_specs=pl.BlockSpec((_RB, dp2), lambda i: (i, 0)),
        out_shape=jax.ShapeDtypeStruct((nh, dp2), jnp.float32),
    )(p1p, g1p, disp, b1p, w2blk)

    p2 = agg(g2p.reshape(n, d_hid), src_t, dst_t, zeros_acc)
    p2p = p2.reshape(_NC, nh, dp2)

    batch2 = batch.astype(jnp.int8).reshape(-1, 1)
    out = pl.pallas_call(
        _final_body,
        grid=(grid,),
        in_specs=[
            pl.BlockSpec((2, _RB, dp2), lambda i: (0, i, 0)),
            pl.BlockSpec((_RB, dp2), lambda i: (i, 0)),
            pl.BlockSpec((_RB, dp2), lambda i: (i, 0)),
            pl.BlockSpec((1, dp2), lambda i: (0, 0)),
            pl.BlockSpec((_RB, 1), lambda i: (i, 0)),
            pl.BlockSpec((_RB, 1), lambda i, g=grid: (i + g, 0)),
            pl.BlockSpec((d_hid, 1), lambda i: (0, 0)),
            pl.BlockSpec((1, 1), lambda i: (0, 0)),
        ],
        out_specs=pl.BlockSpec((g_graphs, 1), lambda i: (0, 0)),
        out_shape=jax.ShapeDtypeStruct((g_graphs, 1), jnp.float32),
        scratch_shapes=[
            pltpu.VMEM((g_graphs, d_hid), jnp.float32),
            pltpu.VMEM((g_graphs, 1), jnp.float32),
        ],
    )(p2p, g2p, disp, b2p, batch2, batch2, Wl, bl.reshape(1, 1))

    return out.reshape(-1)


# speedup vs baseline: 58.0174x; 1.1023x over previous
import functools

import jax
import jax.numpy as jnp
from jax import lax
from jax.experimental import pallas as pl
from jax.experimental.pallas import tpu as pltpu
from jax.experimental.pallas import tpu_sc as plsc

_NC = 2
_NS = 16
_NW = _NC * _NS
_BLK = 128
_NBUF = 8
_LAG = 2
_RB = 1000


def _tile_blocks(w, NB):
    nfull = NB // _NW
    rem = NB % _NW
    base = nfull * w + jnp.minimum(w, rem)
    cnt = nfull + jnp.where(w < rem, 1, 0)
    return base, cnt


def _make_sc_deg(n, RPT, NB, NBT):
    mesh = plsc.VectorSubcoreMesh(core_axis_name="c", subcore_axis_name="s")
    nh = n // 2

    @functools.partial(
        pl.kernel,
        out_type=[
            jax.ShapeDtypeStruct((_NC, n, 16), jnp.float32),
            jax.ShapeDtypeStruct((NB, _BLK), jnp.int32),
            jax.ShapeDtypeStruct((NB, _BLK), jnp.int32),
        ],
        mesh=mesh,
        scratch_types=[
            pltpu.VMEM((NBT, _BLK), jnp.int32),
            pltpu.VMEM((NBT, _BLK), jnp.int32),
            pltpu.VMEM((NBT, _BLK), jnp.int32),
            pltpu.VMEM((_BLK, 16), jnp.float32),
            pltpu.VMEM_SHARED((n, 16), jnp.float32),
            pltpu.SemaphoreType.DMA,
        ],
        compiler_params=pltpu.CompilerParams(use_tc_tiling_on_sc=False),
    )
    def deg_kernel(ei2_hbm, zeros_hbm, out_hbm, srct_hbm, dstt_hbm,
                   sidx, didx, didx_t, ones_b, acc, sem_s):
        c = lax.axis_index("c")
        s = lax.axis_index("s")
        w = c * _NS + s
        base, cnt = _tile_blocks(w, NB)
        pltpu.sync_copy(ei2_hbm.at[pl.ds(base, NBT - 1)],
                        sidx.at[pl.ds(0, NBT - 1)])
        pltpu.sync_copy(ei2_hbm.at[pl.ds(NB + base, NBT - 1)],
                        didx.at[pl.ds(0, NBT - 1)])

        @pl.when(cnt == NBT)
        def _():
            pltpu.sync_copy(ei2_hbm.at[pl.ds(base + NBT - 1, 1)],
                            sidx.at[pl.ds(NBT - 1, 1)])
            pltpu.sync_copy(ei2_hbm.at[pl.ds(NB + base + NBT - 1, 1)],
                            didx.at[pl.ds(NBT - 1, 1)])

        ov = jnp.ones((16,), jnp.float32)

        def fill(r, _):
            ones_b[r, :] = ov
            return 0

        lax.fori_loop(0, _BLK, fill, 0)

        pltpu.sync_copy(zeros_hbm.at[pl.ds(s * RPT, RPT)],
                        acc.at[pl.ds(s * RPT, RPT)])
        plsc.subcore_barrier()

        lag_d = 4

        def body(i, _):
            @pl.when(i >= lag_d)
            def _():
                pltpu.make_async_copy(ones_b, acc.at[didx.at[i - lag_d]],
                                      sem_s).wait()

            pltpu.async_copy(ones_b, acc.at[didx.at[i]], sem_s, add=True)
            for j in range(_BLK // 16):
                v = sidx[i, pl.ds(16 * j, 16)]
                t = v + v
                sidx[i, pl.ds(16 * j, 16)] = jnp.where(v >= nh, t - (n - 1), t)
                u = didx[i, pl.ds(16 * j, 16)]
                t2 = u + u
                didx_t[i, pl.ds(16 * j, 16)] = jnp.where(
                    u >= nh, t2 - (n - 1), t2)
            return 0

        lax.fori_loop(0, cnt, body, 0)

        def drain_deg(k, _):
            i = jnp.maximum(cnt - lag_d, 0) + k
            @pl.when(i < cnt)
            def _():
                pltpu.make_async_copy(ones_b, acc.at[didx.at[i]],
                                      sem_s).wait()
            return 0

        lax.fori_loop(0, lag_d, drain_deg, 0)

        pltpu.sync_copy(sidx.at[pl.ds(0, NBT - 1)],
                        srct_hbm.at[pl.ds(base, NBT - 1)])
        pltpu.sync_copy(didx_t.at[pl.ds(0, NBT - 1)],
                        dstt_hbm.at[pl.ds(base, NBT - 1)])

        @pl.when(cnt == NBT)
        def _():
            pltpu.sync_copy(sidx.at[pl.ds(NBT - 1, 1)],
                            srct_hbm.at[pl.ds(base + NBT - 1, 1)])
            pltpu.sync_copy(didx_t.at[pl.ds(NBT - 1, 1)],
                            dstt_hbm.at[pl.ds(base + NBT - 1, 1)])

        plsc.subcore_barrier()
        pltpu.sync_copy(acc.at[pl.ds(s * RPT, RPT)],
                        out_hbm.at[c].at[pl.ds(s * RPT, RPT)])

    return deg_kernel


def _make_sc_agg(n, RPT, NB, NBT, D):
    mesh = plsc.VectorSubcoreMesh(core_axis_name="c", subcore_axis_name="s")

    @functools.partial(
        pl.kernel,
        out_type=jax.ShapeDtypeStruct((_NC, n, D), jnp.float32),
        mesh=mesh,
        scratch_types=[
            pltpu.VMEM((NBT, _BLK), jnp.int32),
            pltpu.VMEM((NBT, _BLK), jnp.int32),
            pltpu.VMEM((_NBUF, _BLK, D), jnp.float32),
            pltpu.VMEM_SHARED((n, D), jnp.float32),
            pltpu.SemaphoreType.DMA,
            pltpu.SemaphoreType.DMA,
        ],
        compiler_params=pltpu.CompilerParams(use_tc_tiling_on_sc=False),
    )
    def agg_kernel(g_hbm, src_hbm, dst_hbm, zeros_hbm, out_hbm, sidx, didx,
                   rows, acc, sem_g, sem_s):
        c = lax.axis_index("c")
        s = lax.axis_index("s")
        w = c * _NS + s
        base, cnt = _tile_blocks(w, NB)
        pltpu.sync_copy(src_hbm.at[pl.ds(base, NBT - 1)],
                        sidx.at[pl.ds(0, NBT - 1)])
        pltpu.sync_copy(dst_hbm.at[pl.ds(base, NBT - 1)],
                        didx.at[pl.ds(0, NBT - 1)])

        @pl.when(cnt == NBT)
        def _():
            pltpu.sync_copy(src_hbm.at[pl.ds(base + NBT - 1, 1)],
                            sidx.at[pl.ds(NBT - 1, 1)])
            pltpu.sync_copy(dst_hbm.at[pl.ds(base + NBT - 1, 1)],
                            didx.at[pl.ds(NBT - 1, 1)])

        pltpu.sync_copy(zeros_hbm.at[pl.ds(s * RPT, RPT)],
                        acc.at[pl.ds(s * RPT, RPT)])
        plsc.subcore_barrier()

        for b in range(_NBUF):
            pltpu.async_copy(g_hbm.at[sidx.at[b]], rows.at[b], sem_g)

        def body(i, _):
            @pl.when(i >= _LAG)
            def _():
                pltpu.make_async_copy(rows.at[lax.rem(i - _LAG, _NBUF)],
                                      acc.at[didx.at[i - _LAG]], sem_s).wait()
                nxt = i + _NBUF - _LAG

                @pl.when(nxt < cnt)
                def _():
                    pltpu.async_copy(g_hbm.at[sidx.at[nxt]],
                                     rows.at[lax.rem(nxt, _NBUF)], sem_g)

            slot = lax.rem(i, _NBUF)
            pltpu.make_async_copy(g_hbm.at[sidx.at[i]], rows.at[slot],
                                  sem_g).wait()
            pltpu.async_copy(rows.at[slot], acc.at[didx.at[i]], sem_s,
                             add=True)
            return 0

        lax.fori_loop(0, cnt, body, 0)

        def drain(k, _):
            i = cnt - _LAG + k
            pltpu.make_async_copy(rows.at[lax.rem(i, _NBUF)],
                                  acc.at[didx.at[i]], sem_s).wait()
            return 0

        lax.fori_loop(0, _LAG, drain, 0)
        plsc.subcore_barrier()

        pltpu.sync_copy(acc.at[pl.ds(s * RPT, RPT)],
                        out_hbm.at[c].at[pl.ds(s * RPT, RPT)])

    return agg_kernel


def _mm_body(x_ref, w_ref, o_ref):
    o_ref[...] = jnp.dot(x_ref[...], w_ref[...],
                         preferred_element_type=jnp.float32)


def _scale_body(dpl_ref, dpr_ref, h0l_ref, h0r_ref, g_ref, dis_ref):
    degl = dpl_ref[0, :, 0:1] + dpl_ref[1, :, 0:1] + 1.0
    degr = dpr_ref[0, :, 0:1] + dpr_ref[1, :, 0:1] + 1.0
    disl = lax.rsqrt(degl)
    disr = lax.rsqrt(degr)
    b = disl.shape[0]
    dis_ref[...] = jnp.concatenate(
        [jnp.broadcast_to(disl, (b, 64)), jnp.broadcast_to(disr, (b, 64))],
        axis=1)
    g_ref[...] = jnp.concatenate(
        [h0l_ref[...] * disl, h0r_ref[...] * disr], axis=1)


def _layer2_body(p_ref, g1_ref, dis_ref, b1_ref, w2_ref, g2_ref):
    dis = dis_ref[...]
    agg = p_ref[0] + p_ref[1] + g1_ref[...]
    h1 = jnp.maximum(agg * dis + b1_ref[...], 0.0)
    g2_ref[...] = jnp.dot(h1, w2_ref[...],
                          preferred_element_type=jnp.float32) * dis


def _final_body(p_ref, g2_ref, dis_ref, b2_ref, bl_ref2, br_ref2, wl_ref,
                blb_ref, o_ref, acc, cnt):
    i = pl.program_id(0)

    @pl.when(i == 0)
    def _():
        acc[...] = jnp.zeros_like(acc)
        cnt[...] = jnp.zeros_like(cnt)

    agg = p_ref[0] + p_ref[1] + g2_ref[...]
    h2 = jnp.maximum(agg * dis_ref[...] + b2_ref[...], 0.0)
    h2l = h2[:, 0:64]
    h2r = h2[:, 64:128]
    iota = lax.broadcasted_iota(jnp.int32, (1, 64), 1)
    ohl = (bl_ref2[...].astype(jnp.int32) == iota).astype(jnp.float32)
    ohr = (br_ref2[...].astype(jnp.int32) == iota).astype(jnp.float32)
    acc[...] += (
        lax.dot_general(ohl, h2l, (((0,), (0,)), ((), ())),
                        preferred_element_type=jnp.float32)
        + lax.dot_general(ohr, h2r, (((0,), (0,)), ((), ())),
                          preferred_element_type=jnp.float32))
    ones = jnp.ones((ohl.shape[0], 1), jnp.float32)
    cnt[...] += lax.dot_general(ohl + ohr, ones, (((0,), (0,)), ((), ())),
                                preferred_element_type=jnp.float32)

    @pl.when(i == pl.num_programs(0) - 1)
    def _():
        pooled = jnp.dot(acc[...], wl_ref[...],
                         preferred_element_type=jnp.float32)
        o_ref[...] = pooled / jnp.maximum(cnt[...], 1.0) + blb_ref[...]


def kernel(x, edge_index, batch, W1, b1, W2, b2, Wl, bl):
    n, d_in = x.shape
    d_hid = W1.shape[1]
    e = edge_index.shape[1]
    g_graphs = 64
    nh = n // 2
    dp2 = 2 * d_hid

    NB = e // _BLK
    NBT = NB // _NW + 1
    rpt = n // _NS

    ei2 = edge_index.reshape(2 * NB, _BLK)
    zeros_acc = jnp.zeros((n, d_hid), jnp.float32)
    zeros16 = jnp.zeros((n, 16), jnp.float32)

    deg_parts, src_t, dst_t = _make_sc_deg(n, rpt, NB, NBT)(ei2, zeros16)

    grid = nh // _RB
    h0 = pl.pallas_call(
        _mm_body,
        grid=(2 * grid,),
        in_specs=[
            pl.BlockSpec((_RB, d_in), lambda i: (i, 0)),
            pl.BlockSpec((d_in, d_hid), lambda i: (0, 0)),
        ],
        out_specs=pl.BlockSpec((_RB, d_hid), lambda i: (i, 0)),
        out_shape=jax.ShapeDtypeStruct((n, d_hid), jnp.float32),
    )(x, W1)

    g1p, disp = pl.pallas_call(
        _scale_body,
        grid=(grid,),
        in_specs=[
            pl.BlockSpec((2, _RB, 16), lambda i: (0, i, 0)),
            pl.BlockSpec((2, _RB, 16), lambda i, g=grid: (0, i + g, 0)),
            pl.BlockSpec((_RB, d_hid), lambda i: (i, 0)),
            pl.BlockSpec((_RB, d_hid), lambda i, g=grid: (i + g, 0)),
        ],
        out_specs=[
            pl.BlockSpec((_RB, dp2), lambda i: (i, 0)),
            pl.BlockSpec((_RB, dp2), lambda i: (i, 0)),
        ],
        out_shape=[
            jax.ShapeDtypeStruct((nh, dp2), jnp.float32),
            jax.ShapeDtypeStruct((nh, dp2), jnp.float32),
        ],
    )(deg_parts, deg_parts, h0, h0)

    agg = _make_sc_agg(n, rpt, NB, NBT, d_hid)
    p1 = agg(g1p.reshape(n, d_hid), src_t, dst_t, zeros_acc)
    p1p = p1.reshape(_NC, nh, dp2)

    w2blk = jnp.zeros((dp2, dp2), jnp.float32)
    w2blk = w2blk.at[:d_hid, :d_hid].set(W2).at[d_hid:, d_hid:].set(W2)
    b1p = jnp.tile(b1, 2).reshape(1, dp2)
    b2p = jnp.tile(b2, 2).reshape(1, dp2)

    g2p = pl.pallas_call(
        _layer2_body,
        grid=(grid,),
        in_specs=[
            pl.BlockSpec((2, _RB, dp2), lambda i: (0, i, 0)),
            pl.BlockSpec((_RB, dp2), lambda i: (i, 0)),
            pl.BlockSpec((_RB, dp2), lambda i: (i, 0)),
            pl.BlockSpec((1, dp2), lambda i: (0, 0)),
            pl.BlockSpec((dp2, dp2), lambda i: (0, 0)),
        ],
        out_specs=pl.BlockSpec((_RB, dp2), lambda i: (i, 0)),
        out_shape=jax.ShapeDtypeStruct((nh, dp2), jnp.float32),
    )(p1p, g1p, disp, b1p, w2blk)

    p2 = agg(g2p.reshape(n, d_hid), src_t, dst_t, zeros_acc)
    p2p = p2.reshape(_NC, nh, dp2)

    batch2 = batch.astype(jnp.int8).reshape(-1, 1)
    out = pl.pallas_call(
        _final_body,
        grid=(grid,),
        in_specs=[
            pl.BlockSpec((2, _RB, dp2), lambda i: (0, i, 0)),
            pl.BlockSpec((_RB, dp2), lambda i: (i, 0)),
            pl.BlockSpec((_RB, dp2), lambda i: (i, 0)),
            pl.BlockSpec((1, dp2), lambda i: (0, 0)),
            pl.BlockSpec((_RB, 1), lambda i: (i, 0)),
            pl.BlockSpec((_RB, 1), lambda i, g=grid: (i + g, 0)),
            pl.BlockSpec((d_hid, 1), lambda i: (0, 0)),
            pl.BlockSpec((1, 1), lambda i: (0, 0)),
        ],
        out_specs=pl.BlockSpec((g_graphs, 1), lambda i: (0, 0)),
        out_shape=jax.ShapeDtypeStruct((g_graphs, 1), jnp.float32),
        scratch_shapes=[
            pltpu.VMEM((g_graphs, d_hid), jnp.float32),
            pltpu.VMEM((g_graphs, 1), jnp.float32),
        ],
    )(p2p, g2p, disp, b2p, batch2, batch2, Wl, bl.reshape(1, 1))

    return out.reshape(-1)
